# speedup vs baseline: 18.1331x; 18.1331x over previous
import functools

import jax
import jax.numpy as jnp
from jax import lax
from jax.experimental import pallas as pl
from jax.experimental.pallas import tpu as pltpu
from jax.experimental.pallas import tpu_sc as plsc

NC = 2
NS = 16
NW = NC * NS


@functools.cache
def _make_emb(B: int, V: int, D: int, K: int):
    assert B % NW == 0
    bpw = B // NW
    assert bpw % K == 0 and K % 8 == 0
    nsteps = bpw // K

    mesh = plsc.VectorSubcoreMesh(
        core_axis_name="c", subcore_axis_name="s", num_cores=NC, num_subcores=NS
    )

    @functools.partial(
        pl.kernel,
        out_type=jax.ShapeDtypeStruct((B, D), jnp.float32),
        mesh=mesh,
        scratch_types=[
            pltpu.VMEM((bpw,), jnp.int32),
            pltpu.VMEM((K, D), jnp.float32),
            pltpu.SemaphoreType.DMA,
        ],
    )
    def emb(tok_hbm, w_hbm, out_hbm, idx_v, rows_v, gsem):
        wid = lax.axis_index("s") * NC + lax.axis_index("c")
        base = wid * bpw
        pltpu.sync_copy(tok_hbm.at[pl.ds(base, bpw)], idx_v)

        def step_fn(step, carry):
            off = step * K
            idx_sl = idx_v.at[pl.ds(off, K)]
            pltpu.async_copy(w_hbm.at[idx_sl], rows_v, gsem).wait()
            pltpu.sync_copy(rows_v, out_hbm.at[pl.ds(base + off, K)])
            return carry

        lax.fori_loop(0, nsteps, step_fn, 0)

    return emb


def kernel(tok, weight):
    batch, seq = tok.shape
    V, D = weight.shape
    B = batch * seq
    out = _make_emb(B, V, D, 8)(tok.reshape(B), weight)
    return out.reshape(batch, seq, D)


# speedup vs baseline: 20.4942x; 1.1302x over previous
import functools

import jax
import jax.numpy as jnp
from jax import lax
from jax.experimental import pallas as pl
from jax.experimental.pallas import tpu as pltpu
from jax.experimental.pallas import tpu_sc as plsc

NC = 2
NS = 16
NW = NC * NS


@functools.cache
def _make_emb(B: int, V: int, D: int, K: int):
    assert B % NW == 0
    bpw = B // NW
    assert bpw % (2 * K) == 0 and K % 8 == 0
    nout = bpw // (2 * K)

    mesh = plsc.VectorSubcoreMesh(
        core_axis_name="c", subcore_axis_name="s", num_cores=NC, num_subcores=NS
    )

    @functools.partial(
        pl.kernel,
        out_type=jax.ShapeDtypeStruct((B, D), jnp.float32),
        mesh=mesh,
        scratch_types=[
            pltpu.VMEM((bpw,), jnp.int32),
            pltpu.VMEM((K, D), jnp.float32),
            pltpu.VMEM((K, D), jnp.float32),
            pltpu.SemaphoreType.DMA,
            pltpu.SemaphoreType.DMA,
            pltpu.SemaphoreType.DMA,
            pltpu.SemaphoreType.DMA,
        ],
    )
    def emb(tok_hbm, w_hbm, out_hbm, idx_v, rows_a, rows_b,
            gsem_a, gsem_b, ssem_a, ssem_b):
        wid = lax.axis_index("s") * NC + lax.axis_index("c")
        base = wid * bpw
        pltpu.sync_copy(tok_hbm.at[pl.ds(base, bpw)], idx_v)

        def g_start(off, buf, sem):
            pltpu.async_copy(w_hbm.at[idx_v.at[pl.ds(off, K)]], buf, sem)

        def g_wait(buf, sem):
            pltpu.make_async_copy(w_hbm.at[idx_v.at[pl.ds(0, K)]], buf, sem).wait()

        def s_start(off, buf, sem):
            pltpu.async_copy(buf, out_hbm.at[pl.ds(base + off, K)], sem)

        def s_wait(buf, sem):
            pltpu.make_async_copy(buf, out_hbm.at[pl.ds(base, K)], sem).wait()

        g_start(0, rows_a, gsem_a)

        def pair_fn(t, carry):
            off_a = 2 * t * K
            off_b = off_a + K

            @pl.when(t > 0)
            def _():
                s_wait(rows_b, ssem_b)

            g_start(off_b, rows_b, gsem_b)
            g_wait(rows_a, gsem_a)
            s_start(off_a, rows_a, ssem_a)
            g_wait(rows_b, gsem_b)
            s_start(off_b, rows_b, ssem_b)

            @pl.when(t < nout - 1)
            def _():
                s_wait(rows_a, ssem_a)
                g_start(off_a + 2 * K, rows_a, gsem_a)

            return carry

        lax.fori_loop(0, nout, pair_fn, 0)
        s_wait(rows_a, ssem_a)
        s_wait(rows_b, ssem_b)

    return emb


def kernel(tok, weight):
    batch, seq = tok.shape
    V, D = weight.shape
    B = batch * seq
    out = _make_emb(B, V, D, 8)(tok.reshape(B), weight)
    return out.reshape(batch, seq, D)


# speedup vs baseline: 21.5097x; 1.0496x over previous
import functools

import jax
import jax.numpy as jnp
from jax import lax
from jax.experimental import pallas as pl
from jax.experimental.pallas import tpu as pltpu
from jax.experimental.pallas import tpu_sc as plsc

NC = 2
NS = 16
NW = NC * NS


@functools.cache
def _make_emb(B: int, V: int, D: int, K: int):
    RING = 3
    assert B % NW == 0
    bpw = B // NW
    assert bpw % K == 0 and K % 8 == 0
    nsteps = bpw // K
    tail = nsteps % RING
    nloop = nsteps // RING

    mesh = plsc.VectorSubcoreMesh(
        core_axis_name="c", subcore_axis_name="s", num_cores=NC, num_subcores=NS
    )

    @functools.partial(
        pl.kernel,
        out_type=jax.ShapeDtypeStruct((B, D), jnp.float32),
        mesh=mesh,
        scratch_types=[
            pltpu.VMEM((bpw,), jnp.int32),
            [pltpu.VMEM((K, D), jnp.float32)] * RING,
            [pltpu.SemaphoreType.DMA] * RING,
            [pltpu.SemaphoreType.DMA] * RING,
        ],
    )
    def emb(tok_hbm, w_hbm, out_hbm, idx_v, rows, gsems, ssems):
        wid = lax.axis_index("s") * NC + lax.axis_index("c")
        base = wid * bpw
        pltpu.sync_copy(tok_hbm.at[pl.ds(base, bpw)], idx_v)

        def g_start(off, slot):
            pltpu.async_copy(w_hbm.at[idx_v.at[pl.ds(off, K)]], rows[slot],
                             gsems[slot])

        def g_wait(slot):
            pltpu.make_async_copy(w_hbm.at[idx_v.at[pl.ds(0, K)]], rows[slot],
                                  gsems[slot]).wait()

        def s_start(off, slot):
            pltpu.async_copy(rows[slot], out_hbm.at[pl.ds(base + off, K)],
                             ssems[slot])

        def s_wait(slot):
            pltpu.make_async_copy(rows[slot], out_hbm.at[pl.ds(base, K)],
                                  ssems[slot]).wait()

        g_start(0, 0)
        g_start(K, 1)

        def loop_fn(t, carry):
            s0 = t * RING
            for j in range(RING):
                s = s0 + j
                slot = j
                g_wait(slot)
                s_start(s * K, slot)
                if j == 0:
                    @pl.when(t > 0)
                    def _():
                        s_wait(RING - 1)
                else:
                    s_wait(j - 1)
                g_start((s + 2) * K, (j + 2) % RING)
            return carry

        assert tail <= 2
        lax.fori_loop(0, nloop, loop_fn, 0)

        for j in range(tail):
            s = nloop * RING + j
            slot = s % RING
            g_wait(slot)
            s_start(s * K, slot)

        for slot in range(RING):
            s_wait(slot)

    return emb


def kernel(tok, weight):
    batch, seq = tok.shape
    V, D = weight.shape
    B = batch * seq
    out = _make_emb(B, V, D, 8)(tok.reshape(B), weight)
    return out.reshape(batch, seq, D)
